# Initial kernel scaffold; baseline (speedup 1.0000x reference)
import jax
import jax.numpy as jnp
from jax.experimental import pallas as pl


def kernel(x, edge_index, epsilon, W1, b1, g1, be1, W2, b2, g2, be2):
    raise NotImplementedError("write your pallas kernel here")


# speedup vs baseline: 8.4441x; 8.4441x over previous
import functools

import jax
import jax.numpy as jnp
from jax import lax
from jax.experimental import pallas as pl
from jax.experimental.pallas import tpu as pltpu
from jax.experimental.pallas import tpu_sc as plsc


def _make_sc_scatter(n_nodes, d, n_edges, chunk):
  info = plsc.get_sparse_core_info()
  nc, ns = info.num_cores, info.num_subcores
  nw = nc * ns
  edges_per_w = n_edges // nw
  n_chunks = edges_per_w // chunk
  assert edges_per_w % chunk == 0
  assert n_chunks % 8 == 0
  rows_base = (n_nodes // (8 * ns)) * 8
  rows_tail = n_nodes - rows_base * ns

  mesh = plsc.VectorSubcoreMesh(core_axis_name="c", subcore_axis_name="s")

  @functools.partial(
      pl.kernel,
      out_type=jax.ShapeDtypeStruct((nc, n_nodes, d), jnp.float32),
      mesh=mesh,
      scratch_types=[
          pltpu.VMEM((n_chunks, chunk), jnp.int32),
          pltpu.VMEM((n_chunks, chunk), jnp.int32),
          pltpu.VMEM((chunk, d), jnp.float32),
          pltpu.VMEM_SHARED((n_nodes, d), jnp.float32),
          pltpu.SemaphoreType.DMA,
      ],
  )
  def sc_scatter(src_hbm, dst_hbm, x_hbm, out_hbm,
                 src_v, dst_v, rows_v, acc, sem):
    c = lax.axis_index("c")
    s = lax.axis_index("s")
    wid = s * nc + c

    pltpu.sync_copy(src_hbm.at[pl.ds(wid * n_chunks, n_chunks)], src_v)
    pltpu.sync_copy(dst_hbm.at[pl.ds(wid * n_chunks, n_chunks)], dst_v)

    row0 = s * rows_base
    pltpu.sync_copy(x_hbm.at[pl.ds(row0, rows_base)],
                    acc.at[pl.ds(row0, rows_base)])
    if rows_tail:
      @pl.when(s == ns - 1)
      def _():
        pltpu.sync_copy(x_hbm.at[pl.ds(ns * rows_base, rows_tail)],
                        acc.at[pl.ds(ns * rows_base, rows_tail)])
    plsc.subcore_barrier()

    def body(j, carry):
      pltpu.async_copy(x_hbm.at[src_v.at[j]], rows_v, sem).wait()
      pltpu.sync_copy(rows_v, acc.at[dst_v.at[j]], add=True)
      return carry

    lax.fori_loop(0, n_chunks, body, 0)
    plsc.subcore_barrier()

    pltpu.sync_copy(acc.at[pl.ds(row0, rows_base)],
                    out_hbm.at[c].at[pl.ds(row0, rows_base)])
    if rows_tail:
      @pl.when(s == ns - 1)
      def _():
        pltpu.sync_copy(acc.at[pl.ds(ns * rows_base, rows_tail)],
                        out_hbm.at[c].at[pl.ds(ns * rows_base, rows_tail)])

  return sc_scatter


def _bn(h, gamma, beta):
  mean = jnp.mean(h, axis=0, keepdims=True)
  cen = h - mean
  var = jnp.mean(cen * cen, axis=0, keepdims=True)
  return cen * lax.rsqrt(var + 1e-5) * gamma + beta


def _mlp_body(eps_ref, x_ref, p_ref, w1_ref, b1_ref, g1_ref, be1_ref,
              w2_ref, b2_ref, g2_ref, be2_ref, out_ref):
  eps = eps_ref[0, 0]
  combined = (eps - 1.0) * x_ref[...] + p_ref[0] + p_ref[1]
  h = jnp.dot(combined, w1_ref[...], preferred_element_type=jnp.float32)
  h = h + b1_ref[...]
  h = _bn(h, g1_ref[...], be1_ref[...])
  h = jnp.maximum(h, 0.0)
  h = jnp.dot(h, w2_ref[...], preferred_element_type=jnp.float32)
  h = h + b2_ref[...]
  out_ref[...] = _bn(h, g2_ref[...], be2_ref[...])


_CHUNK = 125


@jax.jit
def kernel(x, edge_index, epsilon, W1, b1, g1, be1, W2, b2, g2, be2):
  n_nodes, d = x.shape
  n_edges = edge_index.shape[1]

  sc_scatter = _make_sc_scatter(n_nodes, d, n_edges, _CHUNK)
  nw = 32
  n_chunks = (n_edges // nw) // _CHUNK
  src = edge_index[0].reshape(nw * n_chunks, _CHUNK)
  dst = edge_index[1].reshape(nw * n_chunks, _CHUNK)
  partials = sc_scatter(src, dst, x)

  d_hid = W1.shape[1]
  mlp = pl.pallas_call(
      _mlp_body,
      out_shape=jax.ShapeDtypeStruct((n_nodes, d), jnp.float32),
      in_specs=[
          pl.BlockSpec(memory_space=pltpu.SMEM),
          pl.BlockSpec(memory_space=pltpu.VMEM),
          pl.BlockSpec(memory_space=pltpu.VMEM),
          pl.BlockSpec(memory_space=pltpu.VMEM),
          pl.BlockSpec(memory_space=pltpu.VMEM),
          pl.BlockSpec(memory_space=pltpu.VMEM),
          pl.BlockSpec(memory_space=pltpu.VMEM),
          pl.BlockSpec(memory_space=pltpu.VMEM),
          pl.BlockSpec(memory_space=pltpu.VMEM),
          pl.BlockSpec(memory_space=pltpu.VMEM),
          pl.BlockSpec(memory_space=pltpu.VMEM),
      ],
      out_specs=pl.BlockSpec(memory_space=pltpu.VMEM),
  )
  return mlp(
      jnp.reshape(epsilon.astype(jnp.float32), (1, 1)),
      x, partials, W1,
      jnp.reshape(b1, (1, d_hid)), jnp.reshape(g1, (1, d_hid)),
      jnp.reshape(be1, (1, d_hid)),
      W2, jnp.reshape(b2, (1, d)), jnp.reshape(g2, (1, d)),
      jnp.reshape(be2, (1, d)))


# speedup vs baseline: 11.8943x; 1.4086x over previous
import functools

import jax
import jax.numpy as jnp
from jax import lax
from jax.experimental import pallas as pl
from jax.experimental.pallas import tpu as pltpu
from jax.experimental.pallas import tpu_sc as plsc


def _make_sc_scatter(n_nodes, d, n_edges, chunk):
  info = plsc.get_sparse_core_info()
  nc, ns = info.num_cores, info.num_subcores
  nw = nc * ns
  edges_per_w = n_edges // nw
  n_chunks = edges_per_w // chunk
  assert edges_per_w % chunk == 0
  assert n_chunks % 8 == 0
  rows_base = (n_nodes // (8 * ns)) * 8
  rows_tail = n_nodes - rows_base * ns

  mesh = plsc.VectorSubcoreMesh(core_axis_name="c", subcore_axis_name="s")

  @functools.partial(
      pl.kernel,
      out_type=jax.ShapeDtypeStruct((nc, n_nodes, d), jnp.float32),
      mesh=mesh,
      scratch_types=[
          pltpu.VMEM((n_chunks // 2, chunk), jnp.int32),
          pltpu.VMEM((n_chunks // 2, chunk), jnp.int32),
          pltpu.VMEM((chunk, d), jnp.float32),
          pltpu.VMEM((chunk, d), jnp.float32),
          pltpu.VMEM_SHARED((n_nodes, d), jnp.float32),
          pltpu.SemaphoreType.DMA,
          pltpu.SemaphoreType.DMA,
      ],
  )
  def sc_scatter(src_hbm, dst_hbm, x_hbm, out_hbm,
                 src_v, dst_v, rows0, rows1, acc, sem0, sem1):
    c = lax.axis_index("c")
    s = lax.axis_index("s")
    wid = s * nc + c

    row0 = s * rows_base
    pltpu.sync_copy(x_hbm.at[pl.ds(row0, rows_base)],
                    acc.at[pl.ds(row0, rows_base)])
    if rows_tail:
      @pl.when(s == ns - 1)
      def _():
        pltpu.sync_copy(x_hbm.at[pl.ds(ns * rows_base, rows_tail)],
                        acc.at[pl.ds(ns * rows_base, rows_tail)])
    plsc.subcore_barrier()

    assert n_chunks % 4 == 0
    h_chunks = n_chunks // 2
    for half in range(2):
      pltpu.sync_copy(
          src_hbm.at[pl.ds(wid * n_chunks + half * h_chunks, h_chunks)], src_v)
      pltpu.sync_copy(
          dst_hbm.at[pl.ds(wid * n_chunks + half * h_chunks, h_chunks)], dst_v)
      pltpu.async_copy(x_hbm.at[src_v.at[0]], rows0, sem0)

      def body(i, carry):
        j = 2 * i
        cp1 = pltpu.async_copy(x_hbm.at[src_v.at[j + 1]], rows1, sem1)
        pltpu.make_async_copy(x_hbm.at[src_v.at[j]], rows0, sem0).wait()
        pltpu.sync_copy(rows0, acc.at[dst_v.at[j]], add=True)

        @pl.when(j + 2 < h_chunks)
        def _():
          pltpu.async_copy(x_hbm.at[src_v.at[j + 2]], rows0, sem0)

        cp1.wait()
        pltpu.sync_copy(rows1, acc.at[dst_v.at[j + 1]], add=True)
        return carry

      lax.fori_loop(0, h_chunks // 2, body, 0)
    plsc.subcore_barrier()

    pltpu.sync_copy(acc.at[pl.ds(row0, rows_base)],
                    out_hbm.at[c].at[pl.ds(row0, rows_base)])
    if rows_tail:
      @pl.when(s == ns - 1)
      def _():
        pltpu.sync_copy(acc.at[pl.ds(ns * rows_base, rows_tail)],
                        out_hbm.at[c].at[pl.ds(ns * rows_base, rows_tail)])

  return sc_scatter


def _bn(h, gamma, beta):
  mean = jnp.mean(h, axis=0, keepdims=True)
  cen = h - mean
  var = jnp.mean(cen * cen, axis=0, keepdims=True)
  return cen * lax.rsqrt(var + 1e-5) * gamma + beta


def _mlp_body(eps_ref, x_ref, p_ref, w1_ref, b1_ref, g1_ref, be1_ref,
              w2_ref, b2_ref, g2_ref, be2_ref, out_ref):
  eps = eps_ref[0, 0]
  combined = (eps - 1.0) * x_ref[...] + p_ref[0] + p_ref[1]
  h = jnp.dot(combined, w1_ref[...], preferred_element_type=jnp.float32)
  h = h + b1_ref[...]
  h = _bn(h, g1_ref[...], be1_ref[...])
  h = jnp.maximum(h, 0.0)
  h = jnp.dot(h, w2_ref[...], preferred_element_type=jnp.float32)
  h = h + b2_ref[...]
  out_ref[...] = _bn(h, g2_ref[...], be2_ref[...])


_CHUNK = 125


@jax.jit
def kernel(x, edge_index, epsilon, W1, b1, g1, be1, W2, b2, g2, be2):
  n_nodes, d = x.shape
  n_edges = edge_index.shape[1]

  sc_scatter = _make_sc_scatter(n_nodes, d, n_edges, _CHUNK)
  nw = 32
  n_chunks = (n_edges // nw) // _CHUNK
  src = edge_index[0].reshape(nw * n_chunks, _CHUNK)
  dst = edge_index[1].reshape(nw * n_chunks, _CHUNK)
  partials = sc_scatter(src, dst, x)

  d_hid = W1.shape[1]
  mlp = pl.pallas_call(
      _mlp_body,
      out_shape=jax.ShapeDtypeStruct((n_nodes, d), jnp.float32),
      in_specs=[
          pl.BlockSpec(memory_space=pltpu.SMEM),
          pl.BlockSpec(memory_space=pltpu.VMEM),
          pl.BlockSpec(memory_space=pltpu.VMEM),
          pl.BlockSpec(memory_space=pltpu.VMEM),
          pl.BlockSpec(memory_space=pltpu.VMEM),
          pl.BlockSpec(memory_space=pltpu.VMEM),
          pl.BlockSpec(memory_space=pltpu.VMEM),
          pl.BlockSpec(memory_space=pltpu.VMEM),
          pl.BlockSpec(memory_space=pltpu.VMEM),
          pl.BlockSpec(memory_space=pltpu.VMEM),
          pl.BlockSpec(memory_space=pltpu.VMEM),
      ],
      out_specs=pl.BlockSpec(memory_space=pltpu.VMEM),
  )
  return mlp(
      jnp.reshape(epsilon.astype(jnp.float32), (1, 1)),
      x, partials, W1,
      jnp.reshape(b1, (1, d_hid)), jnp.reshape(g1, (1, d_hid)),
      jnp.reshape(be1, (1, d_hid)),
      W2, jnp.reshape(b2, (1, d)), jnp.reshape(g2, (1, d)),
      jnp.reshape(be2, (1, d)))
